# baseline (speedup 1.0000x reference)
import functools

import jax
import jax.numpy as jnp
from jax import lax
from jax.experimental import pallas as pl
from jax.experimental.pallas import tpu as pltpu
from jax.experimental.pallas import tpu_sc as plsc

_B = 4096
_L = 200
_V = 100000
_EMBED = 64
_N = _B * _L
_NC = 2
_NS = 16
_NW = _NC * _NS
_PER_W = _N // _NW
_W = 128
_T = _PER_W // _W
_NBUF = 4
_PADDED = 128

_C = 2
_M = _N // _C
_CPER_W = _M // _NW
_CT = _CPER_W // _W
_RB = 2048


def _sc_gather(padded, idx_chunk):
    mesh = plsc.VectorSubcoreMesh(core_axis_name="c", subcore_axis_name="s")

    @functools.partial(
        pl.kernel,
        out_type=jax.ShapeDtypeStruct((_M, _PADDED), jnp.float32),
        mesh=mesh,
        scratch_types=[
            pltpu.VMEM((_CPER_W,), jnp.int32),
            *[pltpu.VMEM((_W, _PADDED), jnp.float32) for _ in range(_NBUF)],
            *[pltpu.SemaphoreType.DMA for _ in range(2 * _NBUF)],
        ],
    )
    def gather_kernel(table_hbm, idx_hbm, out_hbm, idx_all, *scratch):
        rows = scratch[:_NBUF]
        gsem = scratch[_NBUF:2 * _NBUF]
        wsem = scratch[2 * _NBUF:]

        wid = lax.axis_index("s") * _NC + lax.axis_index("c")
        base = wid * _CPER_W

        pltpu.sync_copy(idx_hbm.at[pl.ds(base, _CPER_W)], idx_all)

        def gather_start(w, b):
            pltpu.async_copy(
                table_hbm.at[idx_all.at[pl.ds(w * _W, _W)]], rows[b], gsem[b]
            )

        def gather_wait(b):
            pltpu.make_async_copy(
                table_hbm.at[idx_all.at[pl.ds(0, _W)]], rows[b], gsem[b]
            ).wait()

        def write_start(w, b):
            pltpu.async_copy(rows[b], out_hbm.at[pl.ds(base + w * _W, _W)], wsem[b])

        def write_wait(b):
            pltpu.make_async_copy(
                rows[b], out_hbm.at[pl.ds(base, _W)], wsem[b]
            ).wait()

        for b in range(_NBUF):
            gather_start(b, b)

        @pl.loop(0, _CT, step=_NBUF)
        def _(g):
            for b in range(_NBUF):
                gather_wait(b)
                write_start(g + b, b)
            for b in range(_NBUF):
                write_wait(b)

                @pl.when(g + b + _NBUF < _CT)
                def _():
                    gather_start(g + b + _NBUF, b)

    return gather_kernel(padded, idx_chunk)


def _tc_compact(wide):
    def body(wide_ref, out_ref):
        out_ref[...] = wide_ref[:, :_EMBED]

    return pl.pallas_call(
        body,
        grid=(_M // _RB,),
        in_specs=[pl.BlockSpec((_RB, _PADDED), lambda i: (i, 0))],
        out_specs=pl.BlockSpec((_RB, _EMBED), lambda i: (i, 0)),
        out_shape=jax.ShapeDtypeStruct((_M, _EMBED), jnp.float32),
    )(wide)


def kernel(indices, table):
    flat_idx = indices.reshape(_N).astype(jnp.int32)
    padded = jnp.pad(table, ((0, 0), (0, _PADDED - _EMBED)))

    outs = []
    for c in range(_C):
        wide = _sc_gather(padded, flat_idx[c * _M:(c + 1) * _M])
        outs.append(_tc_compact(wide))
    out = jnp.concatenate(outs, axis=0)
    return out.reshape(_B, _L, _EMBED)


# speedup vs baseline: 2.1384x; 2.1384x over previous
import functools

import jax
import jax.numpy as jnp
from jax import lax
from jax.experimental import pallas as pl
from jax.experimental.pallas import tpu as pltpu
from jax.experimental.pallas import tpu_sc as plsc

_B = 4096
_L = 200
_V = 100000
_EMBED = 64
_N = _B * _L
_NC = 2
_NS = 16
_NW = _NC * _NS
_PER_W = _N // _NW
_W = 128
_T = _PER_W // _W
_NBUF = 5
_PADDED = 128


def kernel(indices, table):
    flat_idx = indices.reshape(_N).astype(jnp.int32)
    padded = jnp.pad(table, ((0, 0), (0, _PADDED - _EMBED)))

    mesh = plsc.VectorSubcoreMesh(core_axis_name="c", subcore_axis_name="s")

    @functools.partial(
        pl.kernel,
        out_type=jax.ShapeDtypeStruct((_N, _PADDED), jnp.float32),
        mesh=mesh,
        scratch_types=[
            pltpu.VMEM((_PER_W,), jnp.int32),
            *[pltpu.VMEM((_W, _PADDED), jnp.float32) for _ in range(_NBUF)],
            *[pltpu.SemaphoreType.DMA for _ in range(2 * _NBUF)],
        ],
    )
    def gather_kernel(table_hbm, idx_hbm, out_hbm, idx_all, *scratch):
        rows = scratch[:_NBUF]
        gsem = scratch[_NBUF:2 * _NBUF]
        wsem = scratch[2 * _NBUF:]

        wid = lax.axis_index("s") * _NC + lax.axis_index("c")
        base = wid * _PER_W

        pltpu.sync_copy(idx_hbm.at[pl.ds(base, _PER_W)], idx_all)

        def gather_start(w, b):
            pltpu.async_copy(
                table_hbm.at[idx_all.at[pl.ds(w * _W, _W)]], rows[b], gsem[b]
            )

        def gather_wait(b):
            pltpu.make_async_copy(
                table_hbm.at[idx_all.at[pl.ds(0, _W)]], rows[b], gsem[b]
            ).wait()

        def write_start(w, b):
            pltpu.async_copy(rows[b], out_hbm.at[pl.ds(base + w * _W, _W)], wsem[b])

        def write_wait(b):
            pltpu.make_async_copy(
                rows[b], out_hbm.at[pl.ds(base, _W)], wsem[b]
            ).wait()

        for b in range(_NBUF):
            gather_start(b, b)

        @pl.loop(0, _T, step=_NBUF)
        def _(g):
            for b in range(_NBUF):
                gather_wait(b)
                write_start(g + b, b)
            for b in range(_NBUF):
                write_wait(b)

                @pl.when(g + b + _NBUF < _T)
                def _():
                    gather_start(g + b + _NBUF, b)

    out = gather_kernel(padded, flat_idx)
    return out[:, :_EMBED].reshape(_B, _L, _EMBED)


# speedup vs baseline: 2.1390x; 1.0003x over previous
import functools

import jax
import jax.numpy as jnp
from jax import lax
from jax.experimental import pallas as pl
from jax.experimental.pallas import tpu as pltpu
from jax.experimental.pallas import tpu_sc as plsc

_B = 4096
_L = 200
_V = 100000
_EMBED = 64
_N = _B * _L
_NC = 2
_NS = 16
_NW = _NC * _NS
_PER_W = _N // _NW
_W = 128
_T = _PER_W // _W
_NBUF = 4
_PADDED = 128


def kernel(indices, table):
    flat_idx = indices.reshape(_N).astype(jnp.int32)
    padded = jnp.pad(table, ((0, 0), (0, _PADDED - _EMBED)))

    mesh = plsc.VectorSubcoreMesh(core_axis_name="c", subcore_axis_name="s")

    @functools.partial(
        pl.kernel,
        out_type=jax.ShapeDtypeStruct((_N, _PADDED), jnp.float32),
        mesh=mesh,
        scratch_types=[
            pltpu.VMEM((_PER_W,), jnp.int32),
            *[pltpu.VMEM((_W, _PADDED), jnp.float32) for _ in range(_NBUF)],
            *[pltpu.SemaphoreType.DMA for _ in range(2 * _NBUF)],
        ],
    )
    def gather_kernel(table_hbm, idx_hbm, out_hbm, idx_all, *scratch):
        rows = scratch[:_NBUF]
        gsem = scratch[_NBUF:2 * _NBUF]
        wsem = scratch[2 * _NBUF:]

        wid = lax.axis_index("s") * _NC + lax.axis_index("c")
        base = wid * _PER_W

        pltpu.sync_copy(idx_hbm.at[pl.ds(base, _PER_W)], idx_all)

        def gather_start(w, b):
            pltpu.async_copy(
                table_hbm.at[idx_all.at[pl.ds(w * _W, _W)]], rows[b], gsem[b]
            )

        def gather_wait(b):
            pltpu.make_async_copy(
                table_hbm.at[idx_all.at[pl.ds(0, _W)]], rows[b], gsem[b]
            ).wait()

        def write_start(w, b):
            pltpu.async_copy(rows[b], out_hbm.at[pl.ds(base + w * _W, _W)], wsem[b])

        def write_wait(b):
            pltpu.make_async_copy(
                rows[b], out_hbm.at[pl.ds(base, _W)], wsem[b]
            ).wait()

        for b in range(_NBUF):
            gather_start(b, b)

        @pl.loop(0, _T, step=_NBUF)
        def _(g):
            for b in range(_NBUF):
                gather_wait(b)
                write_start(g + b, b)
            for b in range(_NBUF):
                write_wait(b)

                @pl.when(g + b + _NBUF < _T)
                def _():
                    gather_start(g + b + _NBUF, b)

    out = gather_kernel(padded, flat_idx)
    return out[:, :_EMBED].reshape(_B, _L, _EMBED)
